# baseline (speedup 1.0000x reference)
import functools

import jax
import jax.numpy as jnp
from jax import lax
from jax.experimental import pallas as pl
from jax.experimental.pallas import tpu as pltpu
from jax.experimental.pallas import tpu_sc as plsc

B, C, H, W = 8, 3, 512, 512
NW = 32
ROWS_PER_W = (B * H) // NW
CHR = 8
CH = CHR * W
NCHUNK = ROWS_PER_W // CHR
TBL = 4096 * 3


def _body(flow_hbm, slab_hbm, out_hbm, slab_v,
          tbl0, tbl1, tbl2,
          dx0, dy0, o0, dx1, dy1, o1,
          isem0, osem0, isem1, osem1):
    cid = lax.axis_index("c")
    sid = lax.axis_index("s")
    wid = sid * 2 + cid
    b = wid // 4
    q = wid % 4
    row0 = q * ROWS_PER_W

    kb = b * W
    lane = lax.iota(jnp.int32, 16)
    lane3 = lane * 3

    slots = ((dx0, dy0, o0, isem0, osem0), (dx1, dy1, o1, isem1, osem1))

    def start_in(ci, dxb, dyb, sem):
        r0 = row0 + ci * CHR
        pltpu.async_copy(flow_hbm.at[b, 0, pl.ds(r0, CHR), :], dxb, sem)
        pltpu.async_copy(flow_hbm.at[b, 1, pl.ds(r0, CHR), :], dyb, sem)

    start_in(0, dx0, dy0, isem0)
    pltpu.sync_copy(slab_hbm, slab_v)

    @plsc.parallel_loop(0, TBL // 16, 1, unroll=4)
    def _t(i):
        b16 = i * 16
        idx3 = (lane + b16) * 3
        for c, tbl_c in enumerate((tbl0, tbl1, tbl2)):
            tbl_c[pl.ds(b16, 16)] = plsc.load_gather(slab_v, [idx3 + c])

    @pl.loop(0, NCHUNK, step=2)
    def _pair(ci0):
        for s in range(2):
            dxb, dyb, ob, isem, osem = slots[s]
            ndxb, ndyb, _, nisem, _ = slots[1 - s]
            ci = ci0 + s
            r0 = row0 + ci * CHR

            @pl.when(ci + 1 < NCHUNK)
            def _():
                start_in(ci + 1, ndxb, ndyb, nisem)

            pltpu.make_async_copy(flow_hbm.at[b, 0, pl.ds(0, CHR), :], dxb, isem).wait()
            pltpu.make_async_copy(flow_hbm.at[b, 1, pl.ds(0, CHR), :], dyb, isem).wait()

            @pl.when(ci >= 2)
            def _():
                pltpu.make_async_copy(ob, out_hbm.at[0, pl.ds(0, 3 * CHR), :], osem).wait()

            zv = jnp.zeros((16,), jnp.float32)

            @pl.loop(0, 3 * CHR)
            def _zrow(zr):
                @plsc.parallel_loop(0, W // 16, 1, unroll=4)
                def _zvec(zc):
                    ob[zr, pl.ds(zc * 16, 16)] = zv

            @pl.loop(0, CHR)
            def _row(r):
                h = r0 + r
                hf = lax.broadcast(h, (16,)).astype(jnp.float32)

                @plsc.parallel_loop(0, W // 16, 1, unroll=1)
                def _vec(vc):
                    col = vc * 16
                    dy = dyb[r, pl.ds(col, 16)]
                    y = hf + dy * 64.0
                    act = (y >= 0.0) & (y < 2.0)
                    cnt = plsc.all_reduce_population_count(act)

                    @pl.when(cnt[0] > 0)
                    def _():
                        dx = dxb[r, pl.ds(col, 16)]
                        wv = (lane + col).astype(jnp.float32)
                        x = wv + dx * 64.0
                        xcc = jnp.clip(x, -1.0, 512.0)
                        ycc = jnp.clip(y, -1.0, 3.0)
                        xt = xcc.astype(jnp.int32).astype(jnp.float32)
                        flx = jnp.where(xt > xcc, xt - 1.0, xt)
                        yt = ycc.astype(jnp.int32).astype(jnp.float32)
                        fly = jnp.where(yt > ycc, yt - 1.0, yt)
                        x0c = jnp.clip(flx, 0.0, 511.0)
                        x1c = jnp.clip(flx + 1.0, 0.0, 511.0)
                        y0c = jnp.clip(fly, 0.0, 2.0)
                        y1c = jnp.clip(fly + 1.0, 0.0, 2.0)
                        xw0 = x1c - x
                        xw1 = x - x0c
                        yw0 = y1c - y
                        yw1 = y - y0c
                        ka0 = x0c.astype(jnp.int32) + kb
                        ka1 = x1c.astype(jnp.int32) + kb
                        ya = y0c.astype(jnp.int32) * 4096
                        yb = y1c.astype(jnp.int32) * 4096
                        ra = ya + ka0
                        rb = yb + ka0
                        rc = ya + ka1
                        rd = yb + ka1
                        obase = (r * W + col) * 3
                        for c, tbl_c in enumerate((tbl0, tbl1, tbl2)):
                            ta = plsc.load_gather(tbl_c, [ra])
                            tb = plsc.load_gather(tbl_c, [rb])
                            tc = plsc.load_gather(tbl_c, [rc])
                            td = plsc.load_gather(tbl_c, [rd])
                            m0 = xw0 * ta + xw1 * tc
                            m1 = xw0 * tb + xw1 * td
                            res = yw0 * m0 + yw1 * m1
                            fi = obase + lane3 + c
                            plsc.store_scatter(ob, [fi >> 9, fi & (W - 1)], res)

            pltpu.async_copy(ob, out_hbm.at[b, pl.ds(3 * r0, 3 * CHR), :], osem)

    for s in range(2):
        _, _, ob, _, osem = slots[s]
        pltpu.make_async_copy(ob, out_hbm.at[0, pl.ds(0, 3 * CHR), :], osem).wait()


@functools.partial(jax.jit, donate_argnums=())
def kernel(flow, pqf):
    slab = pqf[0:3, 0, 0:24, :].reshape(-1)

    mesh = plsc.VectorSubcoreMesh(core_axis_name="c", subcore_axis_name="s")
    call = pl.kernel(
        _body,
        out_type=jax.ShapeDtypeStruct((B, C * H, W), jnp.float32),
        mesh=mesh,
        scratch_types=[
            pltpu.VMEM((3 * TBL,), jnp.float32),
            pltpu.VMEM((TBL,), jnp.float32),
            pltpu.VMEM((TBL,), jnp.float32),
            pltpu.VMEM((TBL,), jnp.float32),
            pltpu.VMEM((CHR, W), jnp.float32),
            pltpu.VMEM((CHR, W), jnp.float32),
            pltpu.VMEM((3 * CHR, W), jnp.float32),
            pltpu.VMEM((CHR, W), jnp.float32),
            pltpu.VMEM((CHR, W), jnp.float32),
            pltpu.VMEM((3 * CHR, W), jnp.float32),
            pltpu.SemaphoreType.DMA,
            pltpu.SemaphoreType.DMA,
            pltpu.SemaphoreType.DMA,
            pltpu.SemaphoreType.DMA,
        ],
        compiler_params=pltpu.CompilerParams(needs_layout_passes=False),
    )
    out = call(flow, slab)
    return out.reshape(B, C, H, W)


# speedup vs baseline: 1.2549x; 1.2549x over previous
import functools

import jax
import jax.numpy as jnp
from jax import lax
from jax.experimental import pallas as pl
from jax.experimental.pallas import tpu as pltpu
from jax.experimental.pallas import tpu_sc as plsc

B, C, H, W = 8, 3, 512, 512
NW = 32
ROWS_PER_W = (B * H) // NW
CHR = 8
CH = CHR * W
NCHUNK = ROWS_PER_W // CHR
TBL = 4096 * 3


def _body(flow_hbm, slab_hbm, out_hbm, slab_v,
          tbl0, tbl1, tbl2,
          dx0, dy0, o0, dx1, dy1, o1,
          isem0, osem0, isem1, osem1):
    cid = lax.axis_index("c")
    sid = lax.axis_index("s")
    wid = sid * 2 + cid
    b = wid // 4
    q = wid % 4
    row0 = q * ROWS_PER_W

    kb = b * W
    lane = lax.iota(jnp.int32, 16)
    lane3 = lane * 3

    slots = ((dx0, dy0, o0, isem0, osem0), (dx1, dy1, o1, isem1, osem1))

    def start_in(ci, dxb, dyb, sem):
        r0 = row0 + ci * CHR
        pltpu.async_copy(flow_hbm.at[b, 0, pl.ds(r0, CHR), :], dxb, sem)
        pltpu.async_copy(flow_hbm.at[b, 1, pl.ds(r0, CHR), :], dyb, sem)

    start_in(0, dx0, dy0, isem0)
    pltpu.sync_copy(slab_hbm, slab_v)

    @plsc.parallel_loop(0, TBL // 16, 1, unroll=4)
    def _t(i):
        b16 = i * 16
        idx3 = (lane + b16) * 3
        for c, tbl_c in enumerate((tbl0, tbl1, tbl2)):
            tbl_c[pl.ds(b16, 16)] = plsc.load_gather(slab_v, [idx3 + c])

    @pl.loop(0, NCHUNK, step=2)
    def _pair(ci0):
        for s in range(2):
            dxb, dyb, ob, isem, osem = slots[s]
            ndxb, ndyb, _, nisem, _ = slots[1 - s]
            ci = ci0 + s
            r0 = row0 + ci * CHR

            @pl.when(ci + 1 < NCHUNK)
            def _():
                start_in(ci + 1, ndxb, ndyb, nisem)

            pltpu.make_async_copy(flow_hbm.at[b, 0, pl.ds(0, CHR), :], dxb, isem).wait()
            pltpu.make_async_copy(flow_hbm.at[b, 1, pl.ds(0, CHR), :], dyb, isem).wait()

            @pl.when(ci >= 2)
            def _():
                pltpu.make_async_copy(ob, out_hbm.at[0, pl.ds(0, 3 * CHR), :], osem).wait()

            zv = jnp.zeros((16,), jnp.float32)

            @pl.loop(0, 3 * CHR)
            def _zrow(zr):
                @plsc.parallel_loop(0, W // 16, 1, unroll=4)
                def _zvec(zc):
                    ob[zr, pl.ds(zc * 16, 16)] = zv

            @pl.loop(0, CHR)
            def _row(r):
                h = r0 + r
                hf = lax.broadcast(h, (16,)).astype(jnp.float32)

                @plsc.parallel_loop(0, W // 16, 1, unroll=4,
                                    carry=jnp.zeros((16,), jnp.bool_))
                def _scan(vc, acc):
                    dy = dyb[r, pl.ds(vc * 16, 16)]
                    yv = hf + dy * 64.0
                    return acc | ((yv >= 0.0) & (yv < 2.0))

                cnt = plsc.all_reduce_population_count(_scan)

                @pl.when(cnt[0] > 0)
                def _active_row():
                    @plsc.parallel_loop(0, W // 16, 1, unroll=1)
                    def _vec(vc):
                        col = vc * 16
                        dy = dyb[r, pl.ds(col, 16)]
                        y = hf + dy * 64.0
                        act = (y >= 0.0) & (y < 2.0)
                        dx = dxb[r, pl.ds(col, 16)]
                        wv = (lane + col).astype(jnp.float32)
                        x = wv + dx * 64.0
                        xcc = jnp.clip(x, -1.0, 512.0)
                        ycc = jnp.clip(y, -1.0, 3.0)
                        xt = xcc.astype(jnp.int32).astype(jnp.float32)
                        flx = jnp.where(xt > xcc, xt - 1.0, xt)
                        yt = ycc.astype(jnp.int32).astype(jnp.float32)
                        fly = jnp.where(yt > ycc, yt - 1.0, yt)
                        x0c = jnp.clip(flx, 0.0, 511.0)
                        x1c = jnp.clip(flx + 1.0, 0.0, 511.0)
                        y0c = jnp.clip(fly, 0.0, 2.0)
                        y1c = jnp.clip(fly + 1.0, 0.0, 2.0)
                        xw0 = x1c - x
                        xw1 = x - x0c
                        yw0 = y1c - y
                        yw1 = y - y0c
                        ka0 = x0c.astype(jnp.int32) + kb
                        ka1 = x1c.astype(jnp.int32) + kb
                        ya = y0c.astype(jnp.int32) * 4096
                        yb = y1c.astype(jnp.int32) * 4096
                        ra = ya + ka0
                        rb = yb + ka0
                        rc = ya + ka1
                        rd = yb + ka1
                        obase = (r * W + col) * 3
                        for c, tbl_c in enumerate((tbl0, tbl1, tbl2)):
                            ta = plsc.load_gather(tbl_c, [ra], mask=act)
                            tb = plsc.load_gather(tbl_c, [rb], mask=act)
                            tc = plsc.load_gather(tbl_c, [rc], mask=act)
                            td = plsc.load_gather(tbl_c, [rd], mask=act)
                            m0 = xw0 * ta + xw1 * tc
                            m1 = xw0 * tb + xw1 * td
                            res = yw0 * m0 + yw1 * m1
                            fi = obase + lane3 + c
                            plsc.store_scatter(ob, [fi >> 9, fi & (W - 1)],
                                               res, mask=act)

            pltpu.async_copy(ob, out_hbm.at[b, pl.ds(3 * r0, 3 * CHR), :], osem)

    for s in range(2):
        _, _, ob, _, osem = slots[s]
        pltpu.make_async_copy(ob, out_hbm.at[0, pl.ds(0, 3 * CHR), :], osem).wait()


@functools.partial(jax.jit, donate_argnums=())
def kernel(flow, pqf):
    slab = pqf[0:3, 0, 0:24, :].reshape(-1)

    mesh = plsc.VectorSubcoreMesh(core_axis_name="c", subcore_axis_name="s")
    call = pl.kernel(
        _body,
        out_type=jax.ShapeDtypeStruct((B, C * H, W), jnp.float32),
        mesh=mesh,
        scratch_types=[
            pltpu.VMEM((3 * TBL,), jnp.float32),
            pltpu.VMEM((TBL,), jnp.float32),
            pltpu.VMEM((TBL,), jnp.float32),
            pltpu.VMEM((TBL,), jnp.float32),
            pltpu.VMEM((CHR, W), jnp.float32),
            pltpu.VMEM((CHR, W), jnp.float32),
            pltpu.VMEM((3 * CHR, W), jnp.float32),
            pltpu.VMEM((CHR, W), jnp.float32),
            pltpu.VMEM((CHR, W), jnp.float32),
            pltpu.VMEM((3 * CHR, W), jnp.float32),
            pltpu.SemaphoreType.DMA,
            pltpu.SemaphoreType.DMA,
            pltpu.SemaphoreType.DMA,
            pltpu.SemaphoreType.DMA,
        ],
        compiler_params=pltpu.CompilerParams(needs_layout_passes=False),
    )
    out = call(flow, slab)
    return out.reshape(B, C, H, W)


# speedup vs baseline: 2.2378x; 1.7832x over previous
import functools

import jax
import jax.numpy as jnp
from jax import lax
from jax.experimental import pallas as pl
from jax.experimental.pallas import tpu as pltpu
from jax.experimental.pallas import tpu_sc as plsc

B, C, H, W = 8, 3, 512, 512
NW = 32
ROWS_PER_W = (B * H) // NW
CHR = 8
CH = CHR * W
NCHUNK = ROWS_PER_W // CHR
TBL = 4096 * 3


def _body(flow_hbm, slab_hbm, out_hbm, slab_v,
          tbl0, tbl1, tbl2,
          dx0, dy0, o0, dx1, dy1, o1,
          isem0, osem0, isem1, osem1):
    cid = lax.axis_index("c")
    sid = lax.axis_index("s")
    wid = sid * 2 + cid
    b = wid // 4
    q = wid % 4
    row0 = q * ROWS_PER_W

    kb = b * W
    lane = lax.iota(jnp.int32, 16)
    lane3 = lane * 3

    slots = ((dx0, dy0, o0, isem0, osem0), (dx1, dy1, o1, isem1, osem1))

    def chunk_r0(ci):
        return (q + 4 * ci) * CHR

    def start_in(ci, dxb, dyb, sem):
        r0 = chunk_r0(ci)
        pltpu.async_copy(flow_hbm.at[b, 0, pl.ds(r0, CHR), :], dxb, sem)
        pltpu.async_copy(flow_hbm.at[b, 1, pl.ds(r0, CHR), :], dyb, sem)

    start_in(0, dx0, dy0, isem0)
    pltpu.sync_copy(slab_hbm, slab_v)

    @plsc.parallel_loop(0, TBL // 16, 1, unroll=4)
    def _t(i):
        b16 = i * 16
        idx3 = (lane + b16) * 3
        for c, tbl_c in enumerate((tbl0, tbl1, tbl2)):
            tbl_c[pl.ds(b16, 16)] = plsc.load_gather(slab_v, [idx3 + c])

    @pl.loop(0, NCHUNK, step=2)
    def _pair(ci0):
        for s in range(2):
            dxb, dyb, ob, isem, osem = slots[s]
            ndxb, ndyb, _, nisem, _ = slots[1 - s]
            ci = ci0 + s
            r0 = chunk_r0(ci)

            @pl.when(ci + 1 < NCHUNK)
            def _():
                start_in(ci + 1, ndxb, ndyb, nisem)

            pltpu.make_async_copy(flow_hbm.at[b, 0, pl.ds(0, CHR), :], dxb, isem).wait()
            pltpu.make_async_copy(flow_hbm.at[b, 1, pl.ds(0, CHR), :], dyb, isem).wait()

            @pl.when(ci >= 2)
            def _():
                pltpu.make_async_copy(ob, out_hbm.at[0, pl.ds(0, 3 * CHR), :], osem).wait()

            zv = jnp.zeros((16,), jnp.float32)

            @pl.loop(0, 3 * CHR)
            def _zrow(zr):
                @plsc.parallel_loop(0, W // 16, 1, unroll=4)
                def _zvec(zc):
                    ob[zr, pl.ds(zc * 16, 16)] = zv

            @pl.loop(0, CHR)
            def _row(r):
                h = r0 + r
                hf = lax.broadcast(h, (16,)).astype(jnp.float32)

                @plsc.parallel_loop(0, W // 16, 1, unroll=8,
                                    carry=jnp.zeros((16,), jnp.bool_))
                def _scan(vc, acc):
                    dy = dyb[r, pl.ds(vc * 16, 16)]
                    yv = hf + dy * 64.0
                    return acc | ((yv >= 0.0) & (yv < 2.0))

                cnt = plsc.all_reduce_population_count(_scan)

                @pl.when(cnt[0] > 0)
                def _active_row():
                    @plsc.parallel_loop(0, W // 16, 1, unroll=1)
                    def _vec(vc):
                        col = vc * 16
                        dy = dyb[r, pl.ds(col, 16)]
                        y = hf + dy * 64.0
                        act = (y >= 0.0) & (y < 2.0)
                        dx = dxb[r, pl.ds(col, 16)]
                        wv = (lane + col).astype(jnp.float32)
                        x = wv + dx * 64.0
                        xcc = jnp.clip(x, -1.0, 512.0)
                        ycc = jnp.clip(y, -1.0, 3.0)
                        xt = xcc.astype(jnp.int32).astype(jnp.float32)
                        flx = jnp.where(xt > xcc, xt - 1.0, xt)
                        yt = ycc.astype(jnp.int32).astype(jnp.float32)
                        fly = jnp.where(yt > ycc, yt - 1.0, yt)
                        x0c = jnp.clip(flx, 0.0, 511.0)
                        x1c = jnp.clip(flx + 1.0, 0.0, 511.0)
                        y0c = jnp.clip(fly, 0.0, 2.0)
                        y1c = jnp.clip(fly + 1.0, 0.0, 2.0)
                        xw0 = x1c - x
                        xw1 = x - x0c
                        yw0 = y1c - y
                        yw1 = y - y0c
                        ka0 = x0c.astype(jnp.int32) + kb
                        ka1 = x1c.astype(jnp.int32) + kb
                        ya = y0c.astype(jnp.int32) * 4096
                        yb = y1c.astype(jnp.int32) * 4096
                        ra = ya + ka0
                        rb = yb + ka0
                        rc = ya + ka1
                        rd = yb + ka1
                        obase = (r * W + col) * 3
                        for c, tbl_c in enumerate((tbl0, tbl1, tbl2)):
                            ta = plsc.load_gather(tbl_c, [ra], mask=act)
                            tb = plsc.load_gather(tbl_c, [rb], mask=act)
                            tc = plsc.load_gather(tbl_c, [rc], mask=act)
                            td = plsc.load_gather(tbl_c, [rd], mask=act)
                            m0 = xw0 * ta + xw1 * tc
                            m1 = xw0 * tb + xw1 * td
                            res = yw0 * m0 + yw1 * m1
                            fi = obase + lane3 + c
                            plsc.store_scatter(ob, [fi >> 9, fi & (W - 1)],
                                               res, mask=act)

            pltpu.async_copy(ob, out_hbm.at[b, pl.ds(3 * r0, 3 * CHR), :], osem)

    for s in range(2):
        _, _, ob, _, osem = slots[s]
        pltpu.make_async_copy(ob, out_hbm.at[0, pl.ds(0, 3 * CHR), :], osem).wait()


@functools.partial(jax.jit, donate_argnums=())
def kernel(flow, pqf):
    slab = pqf[0:3, 0, 0:24, :].reshape(-1)

    mesh = plsc.VectorSubcoreMesh(core_axis_name="c", subcore_axis_name="s")
    call = pl.kernel(
        _body,
        out_type=jax.ShapeDtypeStruct((B, C * H, W), jnp.float32),
        mesh=mesh,
        scratch_types=[
            pltpu.VMEM((3 * TBL,), jnp.float32),
            pltpu.VMEM((TBL,), jnp.float32),
            pltpu.VMEM((TBL,), jnp.float32),
            pltpu.VMEM((TBL,), jnp.float32),
            pltpu.VMEM((CHR, W), jnp.float32),
            pltpu.VMEM((CHR, W), jnp.float32),
            pltpu.VMEM((3 * CHR, W), jnp.float32),
            pltpu.VMEM((CHR, W), jnp.float32),
            pltpu.VMEM((CHR, W), jnp.float32),
            pltpu.VMEM((3 * CHR, W), jnp.float32),
            pltpu.SemaphoreType.DMA,
            pltpu.SemaphoreType.DMA,
            pltpu.SemaphoreType.DMA,
            pltpu.SemaphoreType.DMA,
        ],
        compiler_params=pltpu.CompilerParams(needs_layout_passes=False),
    )
    out = call(flow, slab)
    return out.reshape(B, C, H, W)


# speedup vs baseline: 2.3511x; 1.0506x over previous
import functools

import jax
import jax.numpy as jnp
from jax import lax
from jax.experimental import pallas as pl
from jax.experimental.pallas import tpu as pltpu
from jax.experimental.pallas import tpu_sc as plsc

B, C, H, W = 8, 3, 512, 512
NW = 32
ROWS_PER_W = (B * H) // NW
CHR = 8
CH = CHR * W
NCHUNK = ROWS_PER_W // CHR
TBL = 4096 * 3


def _body(flow_hbm, slab_hbm, out_hbm, slab_v,
          tbl0, tbl1, tbl2,
          dx0, dy0, o0, dx1, dy1, o1,
          dirty,
          isem0, osem0, isem1, osem1):
    cid = lax.axis_index("c")
    sid = lax.axis_index("s")
    wid = sid * 2 + cid
    b = wid // 4
    q = wid % 4
    row0 = q * ROWS_PER_W

    kb = b * W
    lane = lax.iota(jnp.int32, 16)
    lane3 = lane * 3

    slots = ((dx0, dy0, o0, isem0, osem0), (dx1, dy1, o1, isem1, osem1))

    def chunk_r0(ci):
        return (q + 4 * ci) * CHR

    def start_in(ci, dxb, dyb, sem):
        r0 = chunk_r0(ci)
        pltpu.async_copy(flow_hbm.at[b, 0, pl.ds(r0, CHR), :], dxb, sem)
        pltpu.async_copy(flow_hbm.at[b, 1, pl.ds(r0, CHR), :], dyb, sem)

    start_in(0, dx0, dy0, isem0)
    dirty[0] = 1
    dirty[1] = 1
    pltpu.sync_copy(slab_hbm, slab_v)

    @plsc.parallel_loop(0, TBL // 16, 1, unroll=4)
    def _t(i):
        b16 = i * 16
        idx3 = (lane + b16) * 3
        for c, tbl_c in enumerate((tbl0, tbl1, tbl2)):
            tbl_c[pl.ds(b16, 16)] = plsc.load_gather(slab_v, [idx3 + c])

    @pl.loop(0, NCHUNK, step=2)
    def _pair(ci0):
        for s in range(2):
            dxb, dyb, ob, isem, osem = slots[s]
            ndxb, ndyb, _, nisem, _ = slots[1 - s]
            ci = ci0 + s
            r0 = chunk_r0(ci)

            @pl.when(ci + 1 < NCHUNK)
            def _():
                start_in(ci + 1, ndxb, ndyb, nisem)

            pltpu.make_async_copy(flow_hbm.at[b, 0, pl.ds(0, CHR), :], dxb, isem).wait()
            pltpu.make_async_copy(flow_hbm.at[b, 1, pl.ds(0, CHR), :], dyb, isem).wait()

            @pl.when(ci >= 2)
            def _():
                pltpu.make_async_copy(ob, out_hbm.at[0, pl.ds(0, 3 * CHR), :], osem).wait()

            zv = jnp.zeros((16,), jnp.float32)

            @pl.when(dirty[s] != 0)
            def _prezero():
                @pl.loop(0, 3 * CHR)
                def _zrow(zr):
                    @plsc.parallel_loop(0, W // 16, 1, unroll=4)
                    def _zvec(zc):
                        ob[zr, pl.ds(zc * 16, 16)] = zv

                dirty[s] = 0

            @pl.loop(0, CHR)
            def _row(r):
                h = r0 + r
                hf = lax.broadcast(h, (16,)).astype(jnp.float32)

                @plsc.parallel_loop(0, W // 16, 1, unroll=8,
                                    carry=jnp.zeros((16,), jnp.bool_))
                def _scan(vc, acc):
                    dy = dyb[r, pl.ds(vc * 16, 16)]
                    yv = hf + dy * 64.0
                    return acc | ((yv >= 0.0) & (yv < 2.0))

                cnt = plsc.all_reduce_population_count(_scan)

                @pl.when(cnt[0] > 0)
                def _active_row():
                    dirty[s] = 1

                    @plsc.parallel_loop(0, W // 16, 1, unroll=1)
                    def _vec(vc):
                        col = vc * 16
                        dy = dyb[r, pl.ds(col, 16)]
                        y = hf + dy * 64.0
                        act = (y >= 0.0) & (y < 2.0)
                        dx = dxb[r, pl.ds(col, 16)]
                        wv = (lane + col).astype(jnp.float32)
                        x = wv + dx * 64.0
                        xcc = jnp.clip(x, -1.0, 512.0)
                        ycc = jnp.clip(y, -1.0, 3.0)
                        xt = xcc.astype(jnp.int32).astype(jnp.float32)
                        flx = jnp.where(xt > xcc, xt - 1.0, xt)
                        yt = ycc.astype(jnp.int32).astype(jnp.float32)
                        fly = jnp.where(yt > ycc, yt - 1.0, yt)
                        x0c = jnp.clip(flx, 0.0, 511.0)
                        x1c = jnp.clip(flx + 1.0, 0.0, 511.0)
                        y0c = jnp.clip(fly, 0.0, 2.0)
                        y1c = jnp.clip(fly + 1.0, 0.0, 2.0)
                        xw0 = x1c - x
                        xw1 = x - x0c
                        yw0 = y1c - y
                        yw1 = y - y0c
                        ka0 = x0c.astype(jnp.int32) + kb
                        ka1 = x1c.astype(jnp.int32) + kb
                        ya = y0c.astype(jnp.int32) * 4096
                        yb = y1c.astype(jnp.int32) * 4096
                        ra = ya + ka0
                        rb = yb + ka0
                        rc = ya + ka1
                        rd = yb + ka1
                        obase = (r * W + col) * 3
                        for c, tbl_c in enumerate((tbl0, tbl1, tbl2)):
                            ta = plsc.load_gather(tbl_c, [ra], mask=act)
                            tb = plsc.load_gather(tbl_c, [rb], mask=act)
                            tc = plsc.load_gather(tbl_c, [rc], mask=act)
                            td = plsc.load_gather(tbl_c, [rd], mask=act)
                            m0 = xw0 * ta + xw1 * tc
                            m1 = xw0 * tb + xw1 * td
                            res = yw0 * m0 + yw1 * m1
                            fi = obase + lane3 + c
                            plsc.store_scatter(ob, [fi >> 9, fi & (W - 1)],
                                               res, mask=act)

            pltpu.async_copy(ob, out_hbm.at[b, pl.ds(3 * r0, 3 * CHR), :], osem)

    for s in range(2):
        _, _, ob, _, osem = slots[s]
        pltpu.make_async_copy(ob, out_hbm.at[0, pl.ds(0, 3 * CHR), :], osem).wait()


@functools.partial(jax.jit, donate_argnums=())
def kernel(flow, pqf):
    slab = pqf[0:3, 0, 0:24, :].reshape(-1)

    mesh = plsc.VectorSubcoreMesh(core_axis_name="c", subcore_axis_name="s")
    call = pl.kernel(
        _body,
        out_type=jax.ShapeDtypeStruct((B, C * H, W), jnp.float32),
        mesh=mesh,
        scratch_types=[
            pltpu.VMEM((3 * TBL,), jnp.float32),
            pltpu.VMEM((TBL,), jnp.float32),
            pltpu.VMEM((TBL,), jnp.float32),
            pltpu.VMEM((TBL,), jnp.float32),
            pltpu.VMEM((CHR, W), jnp.float32),
            pltpu.VMEM((CHR, W), jnp.float32),
            pltpu.VMEM((3 * CHR, W), jnp.float32),
            pltpu.VMEM((CHR, W), jnp.float32),
            pltpu.VMEM((CHR, W), jnp.float32),
            pltpu.VMEM((3 * CHR, W), jnp.float32),
            pltpu.SMEM((2,), jnp.int32),
            pltpu.SemaphoreType.DMA,
            pltpu.SemaphoreType.DMA,
            pltpu.SemaphoreType.DMA,
            pltpu.SemaphoreType.DMA,
        ],
        compiler_params=pltpu.CompilerParams(needs_layout_passes=False),
    )
    out = call(flow, slab)
    return out.reshape(B, C, H, W)
